# Initial kernel scaffold; baseline (speedup 1.0000x reference)
import jax
import jax.numpy as jnp
from jax.experimental import pallas as pl


def kernel(x, W):
    raise NotImplementedError("write your pallas kernel here")


# speedup vs baseline: 1.2345x; 1.2345x over previous
import jax
import jax.numpy as jnp
from jax import lax
from jax.experimental import pallas as pl
from jax.experimental.pallas import tpu as pltpu

_K = 8
_E = 64
_T = 512


def _router_block(x_ref, w_ref, scores_ref, idx_ref):
    x = x_ref[...]
    w = w_ref[...]
    logits = lax.dot_general(
        x, w, (((1,), (1,)), ((), ())), preferred_element_type=jnp.float32
    )
    m = jnp.max(logits, axis=1, keepdims=True)
    s = jnp.sum(jnp.exp(logits - m), axis=1, keepdims=True)

    cols = lax.broadcasted_iota(jnp.int32, logits.shape, 1)
    cur = logits
    svals = []
    sidx = []
    for _ in range(_K):
        mv = jnp.max(cur, axis=1, keepdims=True)
        ii = jnp.min(jnp.where(cur >= mv, cols, _E), axis=1, keepdims=True)
        svals.append(mv)
        sidx.append(ii)
        cur = jnp.where(cols == ii, -jnp.inf, cur)
    top = jnp.concatenate(svals, axis=1)
    scores_ref[...] = jnp.exp(top - m) / s
    idx_ref[...] = jnp.concatenate(sidx, axis=1)


@jax.jit
def kernel(x, W):
    n_tokens, emb = x.shape
    grid = (n_tokens // _T,)
    return pl.pallas_call(
        _router_block,
        grid=grid,
        in_specs=[
            pl.BlockSpec((_T, emb), lambda i: (i, 0)),
            pl.BlockSpec((_E, emb), lambda i: (0, 0)),
        ],
        out_specs=[
            pl.BlockSpec((_T, _K), lambda i: (i, 0)),
            pl.BlockSpec((_T, _K), lambda i: (i, 0)),
        ],
        out_shape=[
            jax.ShapeDtypeStruct((n_tokens, _K), jnp.float32),
            jax.ShapeDtypeStruct((n_tokens, _K), jnp.int32),
        ],
    )(x, W)


# speedup vs baseline: 1.7401x; 1.4095x over previous
import jax
import jax.numpy as jnp
from jax import lax
from jax.experimental import pallas as pl
from jax.experimental.pallas import tpu as pltpu

_K = 8
_E = 64
_T = 512


def _router_block(x_ref, w_ref, scores_ref, idx_ref):
    x = x_ref[...]
    w = w_ref[...]
    logits = lax.dot_general(
        w, x, (((1,), (1,)), ((), ())), preferred_element_type=jnp.float32
    )
    m = jnp.max(logits, axis=0, keepdims=True)
    s = jnp.sum(jnp.exp(logits - m), axis=0, keepdims=True)

    rows = lax.broadcasted_iota(jnp.int32, logits.shape, 0)
    cur = logits
    svals = []
    sidx = []
    for _ in range(_K):
        mv = jnp.max(cur, axis=0, keepdims=True)
        ii = jnp.min(jnp.where(cur >= mv, rows, _E), axis=0, keepdims=True)
        svals.append(mv)
        sidx.append(ii)
        cur = jnp.where(rows == ii, -jnp.inf, cur)
    top = jnp.concatenate(svals, axis=0)
    scores_ref[...] = jnp.exp(top - m) / s
    idx_ref[...] = jnp.concatenate(sidx, axis=0)


@jax.jit
def kernel(x, W):
    n_tokens, emb = x.shape
    grid = (n_tokens // _T,)
    scores_t, idx_t = pl.pallas_call(
        _router_block,
        grid=grid,
        in_specs=[
            pl.BlockSpec((_T, emb), lambda i: (i, 0)),
            pl.BlockSpec((_E, emb), lambda i: (0, 0)),
        ],
        out_specs=[
            pl.BlockSpec((_K, _T), lambda i: (0, i)),
            pl.BlockSpec((_K, _T), lambda i: (0, i)),
        ],
        out_shape=[
            jax.ShapeDtypeStruct((_K, n_tokens), jnp.float32),
            jax.ShapeDtypeStruct((_K, n_tokens), jnp.int32),
        ],
    )(x, W)
    return scores_t.T, idx_t.T


# speedup vs baseline: 1.8647x; 1.0716x over previous
import jax
import jax.numpy as jnp
from jax import lax
from jax.experimental import pallas as pl
from jax.experimental.pallas import tpu as pltpu

_K = 8
_E = 64
_T = 1024


def _router_block(x_ref, w_ref, scores_ref, idx_ref):
    x = x_ref[...]
    w = w_ref[...]
    logits = lax.dot_general(
        w, x, (((1,), (1,)), ((), ())), preferred_element_type=jnp.float32
    )
    m = jnp.max(logits, axis=0, keepdims=True)
    s = jnp.sum(jnp.exp(logits - m), axis=0, keepdims=True)

    rows = lax.broadcasted_iota(jnp.int32, logits.shape, 0)
    cur = logits
    svals = []
    sidx = []
    for _ in range(_K):
        mv = jnp.max(cur, axis=0, keepdims=True)
        ii = jnp.min(jnp.where(cur >= mv, rows, _E), axis=0, keepdims=True)
        svals.append(mv)
        sidx.append(ii)
        cur = jnp.where(rows == ii, -jnp.inf, cur)
    top = jnp.concatenate(svals, axis=0)
    scores_ref[...] = jnp.exp(top - m) / s
    idx_ref[...] = jnp.concatenate(sidx, axis=0)


@jax.jit
def kernel(x, W):
    n_tokens, emb = x.shape
    grid = (n_tokens // _T,)
    scores_t, idx_t = pl.pallas_call(
        _router_block,
        grid=grid,
        in_specs=[
            pl.BlockSpec((_T, emb), lambda i: (i, 0)),
            pl.BlockSpec((_E, emb), lambda i: (0, 0)),
        ],
        out_specs=[
            pl.BlockSpec((_K, _T), lambda i: (0, i)),
            pl.BlockSpec((_K, _T), lambda i: (0, i)),
        ],
        out_shape=[
            jax.ShapeDtypeStruct((_K, n_tokens), jnp.float32),
            jax.ShapeDtypeStruct((_K, n_tokens), jnp.int32),
        ],
    )(x, W)
    return scores_t.T, idx_t.T
